# baseline (speedup 1.0000x reference)
import functools
import jax
import jax.numpy as jnp
from jax import lax
from jax.experimental import pallas as pl
from jax.experimental.pallas import tpu as pltpu
from jax.experimental.pallas import tpu_sc as plsc

_BLK = 2048
_E = 64
_HALF = 32
_K = 8
_T = 16384
_NC = 2
_NS = 16
_L = 16
_NW = _NC * _NS
_TPW = _T // _NW


def _score_kernel(x_ref, w_ref, b_ref, et_ref):
    b = b_ref[...]
    logits = jax.lax.dot_general(
        x_ref[...], w_ref[...], (((1,), (1,)), ((), ())),
        preferred_element_type=jnp.float32) + b
    lt = jax.lax.transpose(logits, (1, 0))
    m = jnp.max(lt, axis=0, keepdims=True)
    et_ref[...] = jnp.exp(lt - m)


def _sc_select(et_hbm, ow_hbm, oi_hbm, ev, wv, iv):
    c = lax.axis_index("c")
    s = lax.axis_index("s")
    wid = s * _NC + c
    base = wid * _TPW
    pltpu.sync_copy(et_hbm.at[:, pl.ds(base, _TPW)], ev)

    low = jnp.full((_L,), -1.0, jnp.float32)

    def body(g, carry):
        t0 = g * _L
        v = [ev[j, pl.ds(t0, _L)] for j in range(_E)]
        e0 = v[:_HALF]
        e1 = v[_HALF:]

        def top2sum(h):
            m1 = h[0]
            for j in range(1, _HALF):
                m1 = jnp.maximum(m1, h[j])
            fi = jnp.full((_L,), _HALF - 1, jnp.int32)
            for j in range(_HALF - 2, -1, -1):
                fi = jnp.where(h[j] == m1, j, fi)
            m2 = low
            for j in range(_HALF):
                m2 = jnp.maximum(m2, jnp.where(fi == j, low, h[j]))
            return m1 + m2

        g0 = top2sum(e0) >= top2sum(e1)
        sv = [jnp.where(g0, e0[j], e1[j]) for j in range(_HALF)]
        base_e = jnp.where(g0, 0, _HALF)

        wk = []
        ik = []
        for _ in range(_K):
            mk = sv[0]
            for j in range(1, _HALF):
                mk = jnp.maximum(mk, sv[j])
            fi = jnp.full((_L,), _HALF - 1, jnp.int32)
            for j in range(_HALF - 2, -1, -1):
                fi = jnp.where(sv[j] == mk, j, fi)
            wk.append(mk)
            ik.append(fi + base_e)
            for j in range(_HALF):
                sv[j] = jnp.where(fi == j, low, sv[j])
        wsum = wk[0]
        for k in range(1, _K):
            wsum = wsum + wk[k]
        wsum = jnp.maximum(wsum, 1e-9)
        for k in range(_K):
            wv[k, pl.ds(t0, _L)] = wk[k] / wsum
            iv[k, pl.ds(t0, _L)] = ik[k]
        return carry

    lax.fori_loop(0, _TPW // _L, body, 0)
    pltpu.sync_copy(wv, ow_hbm.at[:, pl.ds(base, _TPW)])
    pltpu.sync_copy(iv, oi_hbm.at[:, pl.ds(base, _TPW)])


def kernel(x, W, bias):
    Tloc, dim = x.shape
    e = W.shape[0]
    b2 = bias.reshape(1, e)
    et = pl.pallas_call(
        _score_kernel,
        grid=(Tloc // _BLK,),
        in_specs=[
            pl.BlockSpec((_BLK, dim), lambda i: (i, 0)),
            pl.BlockSpec((e, dim), lambda i: (0, 0)),
            pl.BlockSpec((1, e), lambda i: (0, 0)),
        ],
        out_specs=pl.BlockSpec((e, _BLK), lambda i: (0, i)),
        out_shape=jax.ShapeDtypeStruct((e, Tloc), jnp.float32),
    )(x, W, b2)

    sel = pl.kernel(
        _sc_select,
        out_type=[
            jax.ShapeDtypeStruct((_K, Tloc), jnp.float32),
            jax.ShapeDtypeStruct((_K, Tloc), jnp.int32),
        ],
        mesh=plsc.VectorSubcoreMesh(
            core_axis_name="c", subcore_axis_name="s",
            num_cores=_NC, num_subcores=_NS),
        scratch_types=[
            pltpu.VMEM((_E, _TPW), jnp.float32),
            pltpu.VMEM((_K, _TPW), jnp.float32),
            pltpu.VMEM((_K, _TPW), jnp.int32),
        ],
    )
    ow, oi = sel(et)
    return (ow.T.astype(x.dtype), oi.T)


# speedup vs baseline: 1.7168x; 1.7168x over previous
import jax
import jax.numpy as jnp
from jax.experimental import pallas as pl
from jax.experimental.pallas import tpu as pltpu

_BLK = 2048
_E = 64
_HALF = 32
_K = 8


def _gate_kernel(x_ref, w_ref, b_ref, ow_ref, oi_ref):
    b = b_ref[...]
    logits = jax.lax.dot_general(
        x_ref[...], w_ref[...], (((1,), (1,)), ((), ())),
        preferred_element_type=jnp.float32) + b
    lt = jax.lax.transpose(logits, (1, 0))
    R = lt.shape[1]
    m = jnp.max(lt, axis=0, keepdims=True)
    e = jnp.exp(lt - m)
    row = jax.lax.broadcasted_iota(jnp.int32, (_HALF, R), 0)
    neg = jnp.float32(-jnp.inf)

    def top2sum(h):
        m1 = jnp.max(h, axis=0, keepdims=True)
        fi = jnp.min(jnp.where(h == m1, row, _HALF), axis=0, keepdims=True)
        m2 = jnp.max(jnp.where(row == fi, neg, h), axis=0, keepdims=True)
        return m1 + m2

    e0 = e[:_HALF]
    e1 = e[_HALF:]
    rep0 = top2sum(e0)
    rep1 = top2sum(e1)
    g0 = rep0 >= rep1
    s = jnp.where(g0, e0, e1)
    base = jnp.where(g0, 0, _HALF)

    ws = []
    idxs = []
    for _ in range(_K):
        mk = jnp.max(s, axis=0, keepdims=True)
        fi = jnp.min(jnp.where(s == mk, row, _HALF), axis=0, keepdims=True)
        ws.append(mk)
        idxs.append(fi + base)
        s = jnp.where(row == fi, neg, s)
    wmat = jnp.concatenate(ws, axis=0)
    imat = jnp.concatenate(idxs, axis=0)
    wsum = jnp.sum(wmat, axis=0, keepdims=True)
    ow_ref[...] = wmat / jnp.maximum(wsum, 1e-9)
    oi_ref[...] = imat


def kernel(x, W, bias):
    Tloc, dim = x.shape
    e = W.shape[0]
    b2 = bias.reshape(1, e)
    grid = (Tloc // _BLK,)
    ow, oi = pl.pallas_call(
        _gate_kernel,
        grid=grid,
        in_specs=[
            pl.BlockSpec((_BLK, dim), lambda i: (i, 0)),
            pl.BlockSpec((e, dim), lambda i: (0, 0)),
            pl.BlockSpec((1, e), lambda i: (0, 0)),
        ],
        out_specs=[
            pl.BlockSpec((_K, _BLK), lambda i: (0, i)),
            pl.BlockSpec((_K, _BLK), lambda i: (0, i)),
        ],
        out_shape=[
            jax.ShapeDtypeStruct((_K, Tloc), jnp.float32),
            jax.ShapeDtypeStruct((_K, Tloc), jnp.int32),
        ],
    )(x, W, b2)
    return (ow.T.astype(x.dtype), oi.T)
